# baseline (speedup 1.0000x reference)
import functools

import jax
import jax.numpy as jnp
from jax import lax
from jax.experimental import pallas as pl
from jax.experimental.pallas import tpu as pltpu
from jax.experimental.pallas import tpu_sc as plsc

NC = 2
NS = 16
NW = NC * NS


def _pos_encoding(max_len, d_model):
    even_i = jnp.arange(0, d_model, 2, dtype=jnp.float32)
    denominator = jnp.power(10000.0, even_i / d_model)
    position = jnp.arange(max_len, dtype=jnp.float32).reshape(max_len, 1)
    even_pe = jnp.sin(position / denominator)
    odd_pe = jnp.cos(position / denominator)
    return jnp.stack([even_pe, odd_pe], axis=2).reshape(max_len, -1)


@functools.cache
def _make_call(B, L, V, D):
    N = B * L
    LPT = -(-L // NS)
    LP = LPT * NS
    VP = -(-V // 8) * 8
    AUG = LP * VP
    C = 128
    R = 4
    assert N % NW == 0
    RW = N // NW
    assert RW % C == 0
    M = RW // C
    assert M % R == 2 and M > R
    assert D % 16 == 0

    def body(tok_hbm, tab_hbm, pos_hbm, out_hbm, aug_hbm,
             tab_v, pos_v, blk_v, tok_v, idx_v, rows_v,
             sem_b, sem_t, sem_g, sem_o):
        c = lax.axis_index("c")
        s = lax.axis_index("s")
        w = s * NC + c
        row0 = w * RW
        aug_base = c * AUG

        pltpu.async_copy(tok_hbm.at[pl.ds(row0, C)], tok_v.at[0], sem_t)

        pltpu.sync_copy(tab_hbm, tab_v)
        l0 = s * LPT
        pltpu.sync_copy(pos_hbm.at[pl.ds(l0 * D, LPT * D)], pos_v)

        def drain_build():
            pltpu.make_async_copy(
                blk_v.at[0], aug_hbm.at[pl.ds(0, VP)], sem_b).wait()

        def build_one_l(i, b):
            pos_row = [pos_v[pl.ds(i * D + j * 16, 16)] for j in range(D // 16)]
            for v in range(V):
                for j in range(D // 16):
                    blk_v[b, v, pl.ds(j * 16, 16)] = (
                        tab_v[v, pl.ds(j * 16, 16)] + pos_row[j])
            off = (c * LP + l0 + i) * VP
            pltpu.async_copy(blk_v.at[b], aug_hbm.at[pl.ds(off, VP)], sem_b)

        def build_pair(gp, carry):
            for b in range(2):
                i = gp * 2 + b

                @pl.when(i < LPT)
                def _():
                    @pl.when(i >= 2)
                    def _():
                        drain_build()
                    build_one_l(i, b)
            return carry
        lax.fori_loop(0, (LPT + 1) // 2, build_pair, 0)
        for _ in range(min(LPT, 2)):
            drain_build()

        plsc.subcore_barrier()

        def compute_idx(g, b):
            base = row0 + g * C
            for k in range(C // 16):
                tok = tok_v[b, pl.ds(k * 16, 16)]
                flat = lax.iota(jnp.int32, 16) + (base + k * 16)
                l = lax.rem(flat, jnp.full((16,), L, jnp.int32))
                idx_v[b, pl.ds(k * 16, 16)] = tok + l * VP + aug_base

        def fire_tok(g, b):
            pltpu.async_copy(
                tok_hbm.at[pl.ds(row0 + g * C, C)], tok_v.at[b], sem_t)

        def drain_tok(b):
            pltpu.make_async_copy(
                tok_hbm.at[pl.ds(row0, C)], tok_v.at[b], sem_t).wait()

        def fire_gather(b):
            pltpu.async_copy(aug_hbm.at[idx_v.at[b]], rows_v.at[b], sem_g)

        def drain_gather(b):
            pltpu.make_async_copy(
                aug_hbm.at[idx_v.at[b]], rows_v.at[b], sem_g).wait()

        def fire_write(g, b):
            pltpu.async_copy(
                rows_v.at[b], out_hbm.at[pl.ds(row0 + g * C, C)], sem_o)

        def drain_write(b):
            pltpu.make_async_copy(
                rows_v.at[b], out_hbm.at[pl.ds(row0, C)], sem_o).wait()

        for b in range(R):
            fire_tok(b, b)
        drain_tok(0)
        compute_idx(0, 0)
        fire_gather(0)

        def step(g, b):
            drain_gather(b)
            fire_write(g, b)

            @pl.when(g + R < M)
            def _():
                fire_tok(g + R, b)
            drain_tok((b + 1) % R)
            compute_idx(g + 1, (b + 1) % R)

            @pl.when(g >= R - 1)
            def _():
                drain_write((b + 1) % R)
            fire_gather((b + 1) % R)

        def chunk_quad(gq, carry):
            for b in range(R):
                step(gq * R + b, b)
            return carry
        lax.fori_loop(0, (M - 2) // R, chunk_quad, 0)

        g = M - 2
        b = g % R
        drain_gather(b)
        fire_write(g, b)
        drain_tok((b + 1) % R)
        compute_idx(g + 1, (b + 1) % R)
        drain_write((b + 1) % R)
        fire_gather((b + 1) % R)
        g = M - 1
        b = (b + 1) % R
        drain_gather(b)
        fire_write(g, b)
        for bb in range(R):
            drain_write((b + 1 + bb) % R)

    return pl.kernel(
        body,
        out_type=(
            jax.ShapeDtypeStruct((N, D), jnp.float32),
            jax.ShapeDtypeStruct((NC * AUG, D), jnp.float32),
        ),
        mesh=plsc.VectorSubcoreMesh(core_axis_name="c", subcore_axis_name="s"),
        scratch_types=[
            pltpu.VMEM((V, D), jnp.float32),
            pltpu.VMEM((LPT * D,), jnp.float32),
            pltpu.VMEM((2, VP, D), jnp.float32),
            pltpu.VMEM((R, C), jnp.int32),
            pltpu.VMEM((R, C), jnp.int32),
            pltpu.VMEM((R, C, D), jnp.float32),
            pltpu.SemaphoreType.DMA,
            pltpu.SemaphoreType.DMA,
            pltpu.SemaphoreType.DMA,
            pltpu.SemaphoreType.DMA,
        ],
    )


def kernel(tokens, emb_table):
    B, L = tokens.shape
    V, D = emb_table.shape
    call = _make_call(B, L, V, D)
    LPT = -(-L // NS)
    LP = LPT * NS
    pos = _pos_encoding(L, D)
    pos_pad = jnp.zeros((LP, D), jnp.float32).at[:L].set(pos).reshape(-1)
    flat_tokens = tokens.reshape(-1).astype(jnp.int32)
    out, _ = call(flat_tokens, emb_table.astype(jnp.float32), pos_pad)
    return out.reshape(B, L, D)


# speedup vs baseline: 1.1709x; 1.1709x over previous
import functools

import jax
import jax.numpy as jnp
from jax import lax
from jax.experimental import pallas as pl
from jax.experimental.pallas import tpu as pltpu
from jax.experimental.pallas import tpu_sc as plsc

NC = 2
NS = 16
NW = NC * NS


def _pos_encoding(max_len, d_model):
    even_i = jnp.arange(0, d_model, 2, dtype=jnp.float32)
    denominator = jnp.power(10000.0, even_i / d_model)
    position = jnp.arange(max_len, dtype=jnp.float32).reshape(max_len, 1)
    even_pe = jnp.sin(position / denominator)
    odd_pe = jnp.cos(position / denominator)
    return jnp.stack([even_pe, odd_pe], axis=2).reshape(max_len, -1)


@functools.cache
def _make_call(B, L, V, D):
    N = B * L
    LPT = -(-L // NS)
    LP = LPT * NS
    VP = -(-V // 8) * 8
    AUG = LP * VP
    C = 256
    G = C // 128
    assert N % NW == 0
    RW = N // NW
    assert RW % C == 0
    CHUNKS = RW // C
    assert CHUNKS >= 3 and CHUNKS % 2 == 1
    assert D % 16 == 0

    def body(tok_hbm, tab_hbm, pos_hbm, out_hbm, aug_hbm,
             tab_v, pos_v, blk_v, tok_v, idx_v, rows_v,
             sem_b, sem_t, sem_g, sem_o):
        c = lax.axis_index("c")
        s = lax.axis_index("s")
        w = s * NC + c
        row0 = w * RW
        aug_base = c * AUG

        pltpu.async_copy(tok_hbm.at[pl.ds(row0, C)], tok_v.at[0], sem_t)

        pltpu.sync_copy(tab_hbm, tab_v)
        l0 = s * LPT
        pltpu.sync_copy(pos_hbm.at[pl.ds(l0 * D, LPT * D)], pos_v)

        def drain_build():
            pltpu.make_async_copy(
                blk_v.at[0], aug_hbm.at[pl.ds(0, VP)], sem_b).wait()

        def build_one_l(i, b):
            pos_row = [pos_v[pl.ds(i * D + j * 16, 16)] for j in range(D // 16)]
            for v in range(V):
                for j in range(D // 16):
                    blk_v[b, v, pl.ds(j * 16, 16)] = (
                        tab_v[v, pl.ds(j * 16, 16)] + pos_row[j])
            off = (c * LP + l0 + i) * VP
            pltpu.async_copy(blk_v.at[b], aug_hbm.at[pl.ds(off, VP)], sem_b)

        def build_pair(gp, carry):
            for b in range(2):
                i = gp * 2 + b

                @pl.when(i < LPT)
                def _():
                    @pl.when(i >= 2)
                    def _():
                        drain_build()
                    build_one_l(i, b)
            return carry
        lax.fori_loop(0, (LPT + 1) // 2, build_pair, 0)
        for _ in range(min(LPT, 2)):
            drain_build()

        plsc.subcore_barrier()

        def compute_idx(g, b):
            base = row0 + g * C
            for k in range(C // 16):
                tok = tok_v[b, pl.ds(k * 16, 16)]
                flat = lax.iota(jnp.int32, 16) + (base + k * 16)
                l = lax.rem(flat, jnp.full((16,), L, jnp.int32))
                idx_v[b, k // 8, pl.ds((k % 8) * 16, 16)] = (
                    tok + l * VP + aug_base)

        def fire_tok(g, b):
            pltpu.async_copy(
                tok_hbm.at[pl.ds(row0 + g * C, C)], tok_v.at[b], sem_t)

        def drain_tok(b):
            pltpu.make_async_copy(
                tok_hbm.at[pl.ds(row0, C)], tok_v.at[b], sem_t).wait()

        def fire_gather(b):
            for j in range(G):
                pltpu.async_copy(aug_hbm.at[idx_v.at[b, j]],
                                 rows_v.at[b, pl.ds(j * 128, 128)], sem_g)

        def drain_gather(b):
            for j in range(G):
                pltpu.make_async_copy(
                    aug_hbm.at[idx_v.at[b, j]],
                    rows_v.at[b, pl.ds(j * 128, 128)], sem_g).wait()

        def fire_write(g, b):
            for j in range(G):
                pltpu.async_copy(
                    rows_v.at[b, pl.ds(j * 128, 128)],
                    out_hbm.at[pl.ds(row0 + g * C + j * 128, 128)], sem_o)

        def drain_write(b):
            for j in range(G):
                pltpu.make_async_copy(
                    rows_v.at[b, pl.ds(j * 128, 128)],
                    out_hbm.at[pl.ds(row0, 128)], sem_o).wait()

        drain_tok(0)
        compute_idx(0, 0)
        fire_tok(1, 1)
        fire_gather(0)

        def chunk_pair(gp, carry):
            for b in range(2):
                g = gp * 2 + b
                drain_gather(b)
                fire_write(g, b)
                drain_tok(1 - b)
                compute_idx(g + 1, 1 - b)

                @pl.when(g + 2 < CHUNKS)
                def _():
                    fire_tok(g + 2, b)

                @pl.when(g >= 1)
                def _():
                    drain_write(1 - b)
                fire_gather(1 - b)
            return carry
        lax.fori_loop(0, (CHUNKS - 1) // 2, chunk_pair, 0)

        drain_gather(0)
        fire_write(CHUNKS - 1, 0)
        drain_write(1)
        drain_write(0)

    return pl.kernel(
        body,
        out_type=(
            jax.ShapeDtypeStruct((N, D), jnp.float32),
            jax.ShapeDtypeStruct((NC * AUG, D), jnp.float32),
        ),
        mesh=plsc.VectorSubcoreMesh(core_axis_name="c", subcore_axis_name="s"),
        scratch_types=[
            pltpu.VMEM((V, D), jnp.float32),
            pltpu.VMEM((LPT * D,), jnp.float32),
            pltpu.VMEM((2, VP, D), jnp.float32),
            pltpu.VMEM((2, C), jnp.int32),
            pltpu.VMEM((2, G, 128), jnp.int32),
            pltpu.VMEM((2, C, D), jnp.float32),
            pltpu.SemaphoreType.DMA,
            pltpu.SemaphoreType.DMA,
            pltpu.SemaphoreType.DMA,
            pltpu.SemaphoreType.DMA,
        ],
    )


def kernel(tokens, emb_table):
    B, L = tokens.shape
    V, D = emb_table.shape
    call = _make_call(B, L, V, D)
    LPT = -(-L // NS)
    LP = LPT * NS
    pos = _pos_encoding(L, D)
    pos_pad = jnp.zeros((LP, D), jnp.float32).at[:L].set(pos).reshape(-1)
    flat_tokens = tokens.reshape(-1).astype(jnp.int32)
    out, _ = call(flat_tokens, emb_table.astype(jnp.float32), pos_pad)
    return out.reshape(B, L, D)
